# Initial kernel scaffold; baseline (speedup 1.0000x reference)
import jax
import jax.numpy as jnp
from jax.experimental import pallas as pl


def kernel(input_ids, embed_table, lm_head_w, lm_head_b):
    raise NotImplementedError("write your pallas kernel here")


# speedup vs baseline: 3.9594x; 3.9594x over previous
import functools

import jax
import jax.numpy as jnp
from jax import lax
from jax.experimental import pallas as pl
from jax.experimental.pallas import tpu as pltpu
from jax.experimental.pallas import tpu_sc as plsc

NC, NS, L = 2, 16, 16
NW = NC * NS
B, S, V, D = 16384, 200, 10, 4
NTOK = B * S
TOK_PER_W = NTOK // NW
CHUNK = 4096
NCHUNK = TOK_PER_W // CHUNK
GRP = CHUNK // L


def _prep_body(emb_ref, wt_ref, b_ref, tab_ref):
    tab = jnp.dot(emb_ref[...], wt_ref[...],
                  preferred_element_type=jnp.float32)
    tab_ref[...] = tab + b_ref[...]


def _loss_body(part_ref, out_ref):
    out_ref[0, 0] = jnp.sum(part_ref[...]) * (1.0 / (NTOK * V))


def _sc_body(tab_hbm, ids_hbm, out_hbm, part_hbm, tab_v, ids_v, out_v,
             acc_v, sem):
    wid = lax.axis_index("s") * NC + lax.axis_index("c")
    pltpu.sync_copy(tab_hbm, tab_v)
    viota10 = lax.iota(jnp.int32, L) * 10
    base_tok = wid * TOK_PER_W

    def chunk_body(it, vacc):
        tok0 = base_tok + it * CHUNK
        pltpu.sync_copy(ids_hbm.at[pl.ds(tok0, CHUNK)], ids_v)

        def grp_body(g, acc):
            vid16 = ids_v[pl.ds(g * L, L)] * 16
            obase = g * (L * 10)
            for w in range(10):
                vals = plsc.load_gather(tab_v, [vid16 + w])
                plsc.store_scatter(out_v, [viota10 + (obase + w)], vals)
                acc = acc + vals
            return acc

        vacc = lax.fori_loop(0, GRP, grp_body, vacc)
        pltpu.sync_copy(out_v, out_hbm.at[pl.ds(tok0 * 10, CHUNK * 10)])
        return vacc

    vacc = lax.fori_loop(0, NCHUNK, chunk_body, jnp.zeros((L,), jnp.float32))
    acc_v[...] = vacc
    pltpu.sync_copy(acc_v, part_hbm.at[wid])


_sc_expand = functools.partial(
    pl.kernel,
    out_type=(jax.ShapeDtypeStruct((NTOK * 10,), jnp.float32),
              jax.ShapeDtypeStruct((NW, L), jnp.float32)),
    mesh=plsc.VectorSubcoreMesh(core_axis_name="c", subcore_axis_name="s",
                                num_cores=NC, num_subcores=NS),
    scratch_types=(
        pltpu.VMEM((256,), jnp.float32),
        pltpu.VMEM((CHUNK,), jnp.int32),
        pltpu.VMEM((CHUNK * 10,), jnp.float32),
        pltpu.VMEM((L,), jnp.float32),
        pltpu.SemaphoreType.DMA,
    ),
    compiler_params=pltpu.CompilerParams(needs_layout_passes=False),
)(_sc_body)


def kernel(input_ids, embed_table, lm_head_w, lm_head_b):
    emb_p = jnp.zeros((16, 8), jnp.float32).at[:V, :D].set(embed_table)
    wt_p = jnp.zeros((8, 16), jnp.float32).at[:D, :V].set(lm_head_w.T)
    b_p = jnp.zeros((1, 16), jnp.float32).at[0, :V].set(lm_head_b)

    tab = pl.pallas_call(
        _prep_body,
        out_shape=jax.ShapeDtypeStruct((16, 16), jnp.float32),
    )(emb_p, wt_p, b_p)

    ids_flat = input_ids.reshape(NTOK).astype(jnp.int32)
    logits_flat, part = _sc_expand(tab.reshape(256), ids_flat)

    loss = pl.pallas_call(
        _loss_body,
        out_shape=jax.ShapeDtypeStruct((1, 1), jnp.float32),
        out_specs=pl.BlockSpec(memory_space=pltpu.SMEM),
    )(part)[0, 0]

    return (loss, logits_flat.reshape(B, S, V))


# speedup vs baseline: 78.0780x; 19.7196x over previous
import functools

import jax
import jax.numpy as jnp
from jax import lax
from jax.experimental import pallas as pl
from jax.experimental.pallas import tpu as pltpu
from jax.experimental.pallas import tpu_sc as plsc

NC, NS, L = 2, 16, 16
NW = NC * NS
B, S, V, D = 16384, 200, 10, 4
NTOK = B * S
SB = 8
BW = 512
NCHUNK = (S // SB) * (B // BW) // NW
GRP = BW // L

_DIMNUMS = lax.GatherDimensionNumbers(
    offset_dims=(), collapsed_slice_dims=(0,), start_index_map=(0,))


def _lut16(vec, idx):
    return lax.gather(vec, idx[:, None], _DIMNUMS, (1,),
                      mode=lax.GatherScatterMode.PROMISE_IN_BOUNDS)


def _prep_body(w_ref, embt_ref, b_ref, tab_ref):
    tab = jnp.dot(w_ref[...], embt_ref[...],
                  preferred_element_type=jnp.float32)
    tab_ref[...] = tab + b_ref[...]


def _loss_body(part_ref, out_ref):
    out_ref[0, 0] = jnp.sum(part_ref[...]) * (1.0 / (NTOK * V))


def _sc_body(tab_hbm, ids_hbm, out_hbm, part_hbm, tab_v, ids_v, out_v,
             acc_v, sem):
    wid = lax.axis_index("s") * NC + lax.axis_index("c")
    pltpu.sync_copy(tab_hbm, tab_v)
    tcols = [tab_v[v, :] for v in range(V)]

    def chunk_body(it, vacc):
        cid = wid * NCHUNK + it
        st = cid // (B // BW)
        bb = cid % (B // BW)
        s0 = st * SB
        b0 = bb * BW
        pltpu.sync_copy(ids_hbm.at[pl.ds(s0, SB), pl.ds(b0, BW)], ids_v)

        def grp_body(g, acc):
            goff = g * L
            for soff in range(SB):
                vid = ids_v[soff, pl.ds(goff, L)]
                for v in range(V):
                    vals = _lut16(tcols[v], vid)
                    out_v[v, soff, pl.ds(goff, L)] = vals
                    acc = acc + vals
            return acc

        vacc = lax.fori_loop(0, GRP, grp_body, vacc)
        pltpu.sync_copy(out_v,
                        out_hbm.at[:, pl.ds(s0, SB), pl.ds(b0, BW)])
        return vacc

    vacc = lax.fori_loop(0, NCHUNK, chunk_body, jnp.zeros((L,), jnp.float32))
    acc_v[...] = vacc
    pltpu.sync_copy(acc_v, part_hbm.at[wid])


_sc_expand = functools.partial(
    pl.kernel,
    out_type=(jax.ShapeDtypeStruct((V, S, B), jnp.float32),
              jax.ShapeDtypeStruct((NW, L), jnp.float32)),
    mesh=plsc.VectorSubcoreMesh(core_axis_name="c", subcore_axis_name="s",
                                num_cores=NC, num_subcores=NS),
    scratch_types=(
        pltpu.VMEM((16, 16), jnp.float32),
        pltpu.VMEM((SB, BW), jnp.int32),
        pltpu.VMEM((V, SB, BW), jnp.float32),
        pltpu.VMEM((L,), jnp.float32),
        pltpu.SemaphoreType.DMA,
    ),
    compiler_params=pltpu.CompilerParams(needs_layout_passes=False),
)(_sc_body)


def kernel(input_ids, embed_table, lm_head_w, lm_head_b):
    w_p = jnp.zeros((16, 8), jnp.float32).at[:V, :D].set(lm_head_w)
    embt_p = jnp.zeros((8, 16), jnp.float32).at[:D, :V].set(embed_table.T)
    b_p = jnp.zeros((16, 1), jnp.float32).at[:V, 0].set(lm_head_b)

    tab = pl.pallas_call(
        _prep_body,
        out_shape=jax.ShapeDtypeStruct((16, 16), jnp.float32),
    )(w_p, embt_p, b_p)

    ids_t = input_ids.astype(jnp.int32).T
    out_vsb, part = _sc_expand(tab, ids_t)

    loss = pl.pallas_call(
        _loss_body,
        out_shape=jax.ShapeDtypeStruct((1, 1), jnp.float32),
        out_specs=pl.BlockSpec(memory_space=pltpu.SMEM),
    )(part)[0, 0]

    return (loss, out_vsb.transpose(2, 1, 0))


# speedup vs baseline: 126.2001x; 1.6163x over previous
import functools

import jax
import jax.numpy as jnp
from jax import lax
from jax.experimental import pallas as pl
from jax.experimental.pallas import tpu as pltpu
from jax.experimental.pallas import tpu_sc as plsc

NC, NS, L = 2, 16, 16
NW = NC * NS
B, S, V, D = 16384, 200, 10, 4
NTOK = B * S
SB = 8
BW = 256
NBB = B // BW
NCHUNK = (S // SB) * NBB // NW
GRP = BW // L

_DIMNUMS = lax.GatherDimensionNumbers(
    offset_dims=(), collapsed_slice_dims=(0,), start_index_map=(0,))


def _lut16(vec, idx):
    return lax.gather(vec, idx[:, None], _DIMNUMS, (1,),
                      mode=lax.GatherScatterMode.PROMISE_IN_BOUNDS)


def _prep_body(w_ref, embt_ref, b_ref, tab_ref):
    tab = jnp.dot(w_ref[...], embt_ref[...],
                  preferred_element_type=jnp.float32)
    tab_ref[...] = tab + b_ref[...]


def _loss_body(part_ref, out_ref):
    out_ref[0, 0] = jnp.sum(part_ref[...]) * (1.0 / (NTOK * V))


def _sc_body(tab_hbm, ids_hbm, out_hbm, part_hbm, tab_v, ids_v, out_v,
             acc_v, sem_i0, sem_i1, sem_o0, sem_o1):
    wid = lax.axis_index("s") * NC + lax.axis_index("c")
    pltpu.sync_copy(tab_hbm, tab_v)
    tcols = [tab_v[v, :] for v in range(V)]
    sem_ids = (sem_i0, sem_i1)
    sem_out = (sem_o0, sem_o1)
    base = wid * NCHUNK

    def ids_slice(c):
        cid = base + c
        return ids_hbm.at[pl.ds((cid // NBB) * SB, SB),
                          pl.ds((cid % NBB) * BW, BW)]

    def out_slice(c):
        cid = base + c
        return out_hbm.at[:, pl.ds((cid // NBB) * SB, SB),
                          pl.ds((cid % NBB) * BW, BW)]

    pltpu.async_copy(ids_slice(0), ids_v.at[0], sem_ids[0])

    def pair_body(it, vacc):
        for buf in (0, 1):
            c = 2 * it + buf
            cn = jnp.minimum(c + 1, NCHUNK - 1)
            pltpu.async_copy(ids_slice(cn), ids_v.at[1 - buf],
                             sem_ids[1 - buf])
            pltpu.make_async_copy(ids_slice(c), ids_v.at[buf],
                                  sem_ids[buf]).wait()

            @pl.when(it > 0)
            def _wait_out():
                pltpu.make_async_copy(out_v.at[buf], out_slice(c),
                                      sem_out[buf]).wait()

            def grp_body(g, acc):
                goff = g * L
                for soff in range(SB):
                    vid = ids_v[buf, soff, pl.ds(goff, L)]
                    for v in range(V):
                        vals = _lut16(tcols[v], vid)
                        out_v[buf, v, soff, pl.ds(goff, L)] = vals
                        acc = acc + vals
                return acc

            vacc = lax.fori_loop(0, GRP, grp_body, vacc)
            pltpu.async_copy(out_v.at[buf], out_slice(c), sem_out[buf])
        return vacc

    vacc = lax.fori_loop(0, NCHUNK // 2, pair_body,
                         jnp.zeros((L,), jnp.float32))
    for buf in (0, 1):
        pltpu.make_async_copy(out_v.at[buf], out_slice(NCHUNK - 2 + buf),
                              sem_out[buf]).wait()
    pltpu.make_async_copy(ids_slice(NCHUNK - 1), ids_v.at[0],
                          sem_ids[0]).wait()
    acc_v[...] = vacc
    pltpu.sync_copy(acc_v, part_hbm.at[wid])


_sc_expand = functools.partial(
    pl.kernel,
    out_type=(jax.ShapeDtypeStruct((V, S, B), jnp.float32),
              jax.ShapeDtypeStruct((NW, L), jnp.float32)),
    mesh=plsc.VectorSubcoreMesh(core_axis_name="c", subcore_axis_name="s",
                                num_cores=NC, num_subcores=NS),
    scratch_types=(
        pltpu.VMEM((16, 16), jnp.float32),
        pltpu.VMEM((2, SB, BW), jnp.int32),
        pltpu.VMEM((2, V, SB, BW), jnp.float32),
        pltpu.VMEM((L,), jnp.float32),
        pltpu.SemaphoreType.DMA,
        pltpu.SemaphoreType.DMA,
        pltpu.SemaphoreType.DMA,
        pltpu.SemaphoreType.DMA,
    ),
    compiler_params=pltpu.CompilerParams(needs_layout_passes=False),
)(_sc_body)


def kernel(input_ids, embed_table, lm_head_w, lm_head_b):
    w_p = jnp.zeros((16, 8), jnp.float32).at[:V, :D].set(lm_head_w)
    embt_p = jnp.zeros((8, 16), jnp.float32).at[:D, :V].set(embed_table.T)
    b_p = jnp.zeros((16, 1), jnp.float32).at[:V, 0].set(lm_head_b)

    tab = pl.pallas_call(
        _prep_body,
        out_shape=jax.ShapeDtypeStruct((16, 16), jnp.float32),
    )(w_p, embt_p, b_p)

    ids_t = input_ids.astype(jnp.int32).T
    out_vsb, part = _sc_expand(tab, ids_t)

    loss = pl.pallas_call(
        _loss_body,
        out_shape=jax.ShapeDtypeStruct((1, 1), jnp.float32),
        out_specs=pl.BlockSpec(memory_space=pltpu.SMEM),
    )(part)[0, 0]

    return (loss, out_vsb.transpose(2, 1, 0))


# speedup vs baseline: 128.4150x; 1.0176x over previous
import functools

import jax
import jax.numpy as jnp
from jax import lax
from jax.experimental import pallas as pl
from jax.experimental.pallas import tpu as pltpu
from jax.experimental.pallas import tpu_sc as plsc

NC, NS, L = 2, 16, 16
NW = NC * NS
B, S, V, D = 16384, 200, 10, 4
NTOK = B * S
SB = 8
BW = 256
NBB = B // BW
NCHUNK = (S // SB) * NBB // NW
GRP = BW // L

_DIMNUMS = lax.GatherDimensionNumbers(
    offset_dims=(), collapsed_slice_dims=(0,), start_index_map=(0,))


def _lut16(vec, idx):
    return lax.gather(vec, idx[:, None], _DIMNUMS, (1,),
                      mode=lax.GatherScatterMode.PROMISE_IN_BOUNDS)


def _loss_body(part_ref, out_ref):
    out_ref[0, 0] = jnp.sum(part_ref[...]) * (1.0 / (NTOK * V))


def _sc_body(par_hbm, ids_hbm, out_hbm, part_hbm, par_v, ids_v, out_v,
             acc_v, sem_i0, sem_i1, sem_o0, sem_o1):
    wid = lax.axis_index("s") * NC + lax.axis_index("c")
    pltpu.sync_copy(par_hbm, par_v)
    embd = [par_v[d, :] for d in range(D)]
    wd = [par_v[D + d, :] for d in range(D)]
    bvec = par_v[2 * D, :]
    tcols = []
    for v in range(V):
        idxv = jnp.full((L,), v, jnp.int32)
        col = _lut16(bvec, idxv)
        for d in range(D):
            col = col + _lut16(wd[d], idxv) * embd[d]
        tcols.append(col)
    sem_ids = (sem_i0, sem_i1)
    sem_out = (sem_o0, sem_o1)
    base = wid * NCHUNK

    def ids_slice(c):
        cid = base + c
        return ids_hbm.at[pl.ds((cid // NBB) * SB, SB),
                          pl.ds((cid % NBB) * BW, BW)]

    def out_slice(c):
        cid = base + c
        return out_hbm.at[:, pl.ds((cid // NBB) * SB, SB),
                          pl.ds((cid % NBB) * BW, BW)]

    pltpu.async_copy(ids_slice(0), ids_v.at[0], sem_ids[0])

    def pair_body(it, vacc):
        for buf in (0, 1):
            c = 2 * it + buf
            cn = jnp.minimum(c + 1, NCHUNK - 1)
            pltpu.async_copy(ids_slice(cn), ids_v.at[1 - buf],
                             sem_ids[1 - buf])
            pltpu.make_async_copy(ids_slice(c), ids_v.at[buf],
                                  sem_ids[buf]).wait()

            @pl.when(it > 0)
            def _wait_out():
                pltpu.make_async_copy(out_v.at[buf], out_slice(c),
                                      sem_out[buf]).wait()

            def grp_body(g, acc):
                goff = g * L
                for soff in range(SB):
                    vid = ids_v[buf, soff, pl.ds(goff, L)]
                    for v in range(V):
                        vals = _lut16(tcols[v], vid)
                        out_v[buf, v, soff, pl.ds(goff, L)] = vals
                        acc = acc + vals
                return acc

            vacc = lax.fori_loop(0, GRP, grp_body, vacc)
            pltpu.async_copy(out_v.at[buf], out_slice(c), sem_out[buf])
        return vacc

    vacc = lax.fori_loop(0, NCHUNK // 2, pair_body,
                         jnp.zeros((L,), jnp.float32))
    for buf in (0, 1):
        pltpu.make_async_copy(out_v.at[buf], out_slice(NCHUNK - 2 + buf),
                              sem_out[buf]).wait()
    pltpu.make_async_copy(ids_slice(NCHUNK - 1), ids_v.at[0],
                          sem_ids[0]).wait()
    acc_v[...] = vacc
    pltpu.sync_copy(acc_v, part_hbm.at[wid])


_sc_expand = functools.partial(
    pl.kernel,
    out_type=(jax.ShapeDtypeStruct((V, S, B), jnp.float32),
              jax.ShapeDtypeStruct((NW, L), jnp.float32)),
    mesh=plsc.VectorSubcoreMesh(core_axis_name="c", subcore_axis_name="s",
                                num_cores=NC, num_subcores=NS),
    scratch_types=(
        pltpu.VMEM((16, 16), jnp.float32),
        pltpu.VMEM((2, SB, BW), jnp.int32),
        pltpu.VMEM((2, V, SB, BW), jnp.float32),
        pltpu.VMEM((L,), jnp.float32),
        pltpu.SemaphoreType.DMA,
        pltpu.SemaphoreType.DMA,
        pltpu.SemaphoreType.DMA,
        pltpu.SemaphoreType.DMA,
    ),
    compiler_params=pltpu.CompilerParams(needs_layout_passes=False),
)(_sc_body)


def kernel(input_ids, embed_table, lm_head_w, lm_head_b):
    par = (jnp.zeros((16, 16), jnp.float32)
           .at[:D, :V].set(embed_table.T)
           .at[D:2 * D, :V].set(lm_head_w.T)
           .at[2 * D, :V].set(lm_head_b))

    ids_t = input_ids.astype(jnp.int32).T
    out_vsb, part = _sc_expand(par, ids_t)

    loss = pl.pallas_call(
        _loss_body,
        out_shape=jax.ShapeDtypeStruct((1, 1), jnp.float32),
        out_specs=pl.BlockSpec(memory_space=pltpu.SMEM),
    )(part)[0, 0]

    return (loss, out_vsb.transpose(2, 1, 0))
